# Initial kernel scaffold; baseline (speedup 1.0000x reference)
import jax
import jax.numpy as jnp
from jax.experimental import pallas as pl


def kernel(x_s, x_t, W1, b1, W2, b2, fcW1, fcb1, fcW2, fcb2, dW, db, edge_index_s, edge_index_t, batch_s, batch_t, labels_s):
    raise NotImplementedError("write your pallas kernel here")


# speedup vs baseline: 19.1609x; 19.1609x over previous
import functools

import jax
import jax.numpy as jnp
from jax import lax
from jax.experimental import pallas as pl
from jax.experimental.pallas import tpu as pltpu
from jax.experimental.pallas import tpu_sc as plsc

N = 10000
D = 128
G = 256
C = 10
E = 320000

NPAD = 10112
RPT = NPAD // 16
CH = 128
NCH = 160
ET = NCH * CH
EPAD = 16 * ET
IB = 32
NIB = NCH // IB
RB = 2 * NPAD // 8
GSEG = 2 * G


@functools.cache
def _sc_kernels():
    mesh = plsc.VectorSubcoreMesh(core_axis_name="c", subcore_axis_name="s")

    @functools.partial(
        pl.kernel,
        mesh=mesh,
        out_type=jax.ShapeDtypeStruct((2 * NPAD, D), jnp.float32),
        scratch_types=[
            pltpu.VMEM((IB, CH), jnp.int32),
            pltpu.VMEM((CH, D), jnp.float32),
            pltpu.VMEM_SHARED((NPAD, D), jnp.float32),
        ],
    )
    def sc_degree(dst_hbm, ones_hbm, zeros_hbm, out_hbm, dst_v, ones_v, acc):
        c = lax.axis_index("c")
        s = lax.axis_index("s")
        pltpu.sync_copy(zeros_hbm, acc.at[pl.ds(s * RPT, RPT)])
        pltpu.sync_copy(ones_hbm, ones_v)
        plsc.subcore_barrier()

        def block(bi, carry):
            base = (c * 16 + s) * NCH + bi * IB
            pltpu.sync_copy(dst_hbm.at[pl.ds(base, IB)], dst_v)

            def chunk(j, carry2):
                pltpu.sync_copy(ones_v, acc.at[dst_v.at[j]], add=True)
                return carry2

            lax.fori_loop(0, IB, chunk, 0)
            return carry

        lax.fori_loop(0, NIB, block, 0)
        plsc.subcore_barrier()
        pltpu.sync_copy(acc.at[pl.ds(s * RPT, RPT)],
                        out_hbm.at[pl.ds(c * NPAD + s * RPT, RPT)])

    @functools.partial(
        pl.kernel,
        mesh=mesh,
        out_type=jax.ShapeDtypeStruct((2 * NPAD, D), jnp.float32),
        scratch_types=[
            pltpu.VMEM((IB, CH), jnp.int32),
            pltpu.VMEM((IB, CH), jnp.int32),
            pltpu.VMEM((CH, D), jnp.float32),
            pltpu.VMEM((CH, D), jnp.float32),
            pltpu.VMEM_SHARED((NPAD, D), jnp.float32),
            pltpu.SemaphoreType.DMA,
            pltpu.SemaphoreType.DMA,
        ],
    )
    def sc_edge_pass(y_hbm, src_hbm, dst_hbm, zeros_hbm, out_hbm,
                     src_v, dst_v, rows0, rows1, acc, sem0, sem1):
        c = lax.axis_index("c")
        s = lax.axis_index("s")
        pltpu.sync_copy(zeros_hbm, acc.at[pl.ds(s * RPT, RPT)])
        plsc.subcore_barrier()

        def block(bi, carry):
            base = (c * 16 + s) * NCH + bi * IB
            pltpu.sync_copy(src_hbm.at[pl.ds(base, IB)], src_v)
            pltpu.sync_copy(dst_hbm.at[pl.ds(base, IB)], dst_v)

            def pair(i, carry2):
                j0 = 2 * i
                j1 = 2 * i + 1
                pltpu.async_copy(y_hbm.at[src_v.at[j0]], rows0, sem0)
                pltpu.async_copy(y_hbm.at[src_v.at[j1]], rows1, sem1)
                pltpu.make_async_copy(y_hbm.at[src_v.at[j0]], rows0, sem0).wait()
                pltpu.sync_copy(rows0, acc.at[dst_v.at[j0]], add=True)
                pltpu.make_async_copy(y_hbm.at[src_v.at[j1]], rows1, sem1).wait()
                pltpu.sync_copy(rows1, acc.at[dst_v.at[j1]], add=True)
                return carry2

            lax.fori_loop(0, IB // 2, pair, 0)
            return carry

        lax.fori_loop(0, NIB, block, 0)
        plsc.subcore_barrier()
        pltpu.sync_copy(acc.at[pl.ds(s * RPT, RPT)],
                        out_hbm.at[pl.ds(c * NPAD + s * RPT, RPT)])

    return sc_degree, sc_edge_pass


def _sc_degree(dst_pack, onesD, zerosD):
    return _sc_kernels()[0](dst_pack, onesD, zerosD)


def _sc_edge_pass(y, src_pack, dst_pack, zerosD):
    return _sc_kernels()[1](y, src_pack, dst_pack, zerosD)


def _scale_body(x_ref, deg_ref, y_ref):
    dinv = lax.rsqrt(deg_ref[:, 0:1] + 1.0)
    y_ref[...] = x_ref[...] * dinv


def _mid_body(z_ref, y0_ref, deg_ref, b_ref, w_ref, bias_ref,
              y1_ref, ps_ref, cnt_ref):
    i = pl.program_id(0)
    dinv = lax.rsqrt(deg_ref[:, 0:1] + 1.0)
    a1 = dinv * (z_ref[...] + y0_ref[...])
    f1 = jnp.dot(a1, w_ref[...], preferred_element_type=jnp.float32) + bias_ref[...]
    y1_ref[...] = dinv * f1
    onehot = (b_ref[...] == lax.broadcasted_iota(jnp.int32, (RB, GSEG), 1)
              ).astype(jnp.float32)
    dn = (((0,), (0,)), ((), ()))
    blk_ps = lax.dot_general(onehot, f1, dn, preferred_element_type=jnp.float32)
    blk_cnt = lax.dot_general(onehot, jnp.ones_like(f1), dn,
                              preferred_element_type=jnp.float32)

    @pl.when(i == 0)
    def _():
        ps_ref[...] = blk_ps
        cnt_ref[...] = blk_cnt

    @pl.when(i > 0)
    def _():
        ps_ref[...] += blk_ps
        cnt_ref[...] += blk_cnt


def _final_body(z_ref, y1_ref, deg_ref, b_ref, w_ref, bias_ref,
                ps1_ref, cnt_ref, fcw1_ref, fcb1_ref, fcw2_ref, fcb2_ref,
                dw1_ref, dw2_ref, dwb_ref, db_ref, lab_ref,
                ps2_ref, loss_ref, closs_ref, dloss_ref):
    i = pl.program_id(0)
    dinv = lax.rsqrt(deg_ref[:, 0:1] + 1.0)
    a2 = dinv * (z_ref[...] + y1_ref[...])
    f2 = jnp.dot(a2, w_ref[...], preferred_element_type=jnp.float32) + bias_ref[...]
    onehot = (b_ref[...] == lax.broadcasted_iota(jnp.int32, (RB, GSEG), 1)
              ).astype(jnp.float32)
    dn = (((0,), (0,)), ((), ()))
    blk_ps = lax.dot_general(onehot, f2, dn, preferred_element_type=jnp.float32)

    @pl.when(i == 0)
    def _():
        ps2_ref[...] = blk_ps

    @pl.when(i > 0)
    def _():
        ps2_ref[...] += blk_ps

    @pl.when(i == 7)
    def _():
        cnt = jnp.maximum(cnt_ref[...], 1.0)
        p1 = ps1_ref[...] / cnt
        p2 = ps2_ref[...] / cnt
        p1s, p1t = p1[:G], p1[G:]
        p2s, p2t = p2[:G], p2[G:]

        def head(p):
            h = jnp.maximum(
                jnp.dot(p, fcw1_ref[...], preferred_element_type=jnp.float32)
                + fcb1_ref[...], 0.0)
            return (jnp.dot(h, fcw2_ref[...], preferred_element_type=jnp.float32)
                    + fcb2_ref[...])

        fsb = head(p2s)
        ftb = head(p2t)
        y16 = (lab_ref[...] == lax.broadcasted_iota(jnp.int32, (G, 16), 1)
               ).astype(jnp.float32)
        z10 = fsb[:, :C]
        y10 = y16[:, :C]
        closs = jnp.mean(jnp.maximum(z10, 0.0) - z10 * y10
                         + jnp.log1p(jnp.exp(-jnp.abs(z10))))

        def dom_logits(pa, pb, fb):
            return (jnp.dot(pa, dw1_ref[...], preferred_element_type=jnp.float32)
                    + jnp.dot(pb, dw2_ref[...], preferred_element_type=jnp.float32)
                    + jnp.dot(fb, dwb_ref[...], preferred_element_type=jnp.float32)
                    + db_ref[...])

        dp_s = dom_logits(p1s, p2s, fsb)
        dp_t = dom_logits(p1t, p2t, ftb)

        def lse2(dp):
            m = jnp.maximum(dp[:, 0:1], dp[:, 1:2])
            return m + jnp.log(jnp.exp(dp[:, 0:1] - m) + jnp.exp(dp[:, 1:2] - m))

        dloss = -(jnp.sum(dp_s[:, 0:1] - lse2(dp_s))
                  + jnp.sum(dp_t[:, 1:2] - lse2(dp_t))) / (2.0 * G)
        loss = closs + 0.01 * dloss
        loss_ref[...] = loss.reshape(1, 1)
        closs_ref[...] = closs.reshape(1, 1)
        dloss_ref[...] = dloss.reshape(1, 1)


def _row_spec(w):
    return pl.BlockSpec((RB, w), lambda i: (i, 0))


def _const_spec(shape):
    return pl.BlockSpec(shape, lambda i: tuple(0 for _ in shape))


def _tc_scale(x_pack, deg16):
    return pl.pallas_call(
        _scale_body,
        grid=(8,),
        in_specs=[_row_spec(D), _row_spec(D)],
        out_specs=_row_spec(D),
        out_shape=jax.ShapeDtypeStruct((2 * NPAD, D), jnp.float32),
    )(x_pack, deg16)


def _tc_mid(z1, y0, deg16, batch_pack, w1, b1r):
    return pl.pallas_call(
        _mid_body,
        grid=(8,),
        in_specs=[_row_spec(D), _row_spec(D), _row_spec(D), _row_spec(1),
                  _const_spec((D, D)), _const_spec((1, D))],
        out_specs=[_row_spec(D), _const_spec((GSEG, D)), _const_spec((GSEG, D))],
        out_shape=[jax.ShapeDtypeStruct((2 * NPAD, D), jnp.float32),
                   jax.ShapeDtypeStruct((GSEG, D), jnp.float32),
                   jax.ShapeDtypeStruct((GSEG, D), jnp.float32)],
    )(z1, y0, deg16, batch_pack, w1, b1r)


def _tc_final(z2, y1, deg16, batch_pack, w2, b2r, ps1, cnt,
              fcw1, fcb1r, fcw2p, fcb2p, dw1, dw2, dwbp, dbr, lab):
    return pl.pallas_call(
        _final_body,
        grid=(8,),
        in_specs=[_row_spec(D), _row_spec(D), _row_spec(D), _row_spec(1),
                  _const_spec((D, D)), _const_spec((1, D)),
                  _const_spec((GSEG, D)), _const_spec((GSEG, D)),
                  _const_spec((D, 16)), _const_spec((1, 16)),
                  _const_spec((16, 16)), _const_spec((1, 16)),
                  _const_spec((D, 2)), _const_spec((D, 2)),
                  _const_spec((16, 2)), _const_spec((1, 2)),
                  _const_spec((G, 1))],
        out_specs=[_const_spec((GSEG, D)), _const_spec((1, 1)),
                   _const_spec((1, 1)), _const_spec((1, 1))],
        out_shape=[jax.ShapeDtypeStruct((GSEG, D), jnp.float32),
                   jax.ShapeDtypeStruct((1, 1), jnp.float32),
                   jax.ShapeDtypeStruct((1, 1), jnp.float32),
                   jax.ShapeDtypeStruct((1, 1), jnp.float32)],
    )(z2, y1, deg16, batch_pack, w2, b2r, ps1, cnt,
      fcw1, fcb1r, fcw2p, fcb2p, dw1, dw2, dwbp, dbr, lab)


def _pad_edges(idx, row_offset):
    pad = N + (jnp.arange(EPAD - E, dtype=jnp.int32) % 16)
    full = jnp.concatenate([idx.astype(jnp.int32), pad]) + row_offset
    return full.reshape(16 * NCH, CH)


def kernel(x_s, x_t, W1, b1, W2, b2, fcW1, fcb1, fcW2, fcb2, dW, db,
           edge_index_s, edge_index_t, batch_s, batch_t, labels_s):
    f32 = jnp.float32
    zpad = jnp.zeros((NPAD - N, D), f32)
    x_pack = jnp.concatenate([x_s.astype(f32), zpad, x_t.astype(f32), zpad])

    src_pack = jnp.concatenate([_pad_edges(edge_index_s[0], 0),
                                _pad_edges(edge_index_t[0], NPAD)])
    dst_pack = jnp.concatenate([_pad_edges(edge_index_s[1], 0),
                                _pad_edges(edge_index_t[1], 0)])

    bpad = jnp.full((NPAD - N,), GSEG, jnp.int32)
    batch_pack = jnp.concatenate([
        batch_s.astype(jnp.int32), bpad,
        batch_t.astype(jnp.int32) + G, bpad]).reshape(2 * NPAD, 1)

    onesD = jnp.ones((CH, D), f32)
    zerosD = jnp.zeros((RPT, D), f32)

    deg16 = _sc_degree(dst_pack, onesD, zerosD)
    y0 = _tc_scale(x_pack, deg16)
    z1 = _sc_edge_pass(y0, src_pack, dst_pack, zerosD)

    b1r = b1.astype(f32).reshape(1, D)
    y1, ps1, cnt = _tc_mid(z1, y0, deg16, batch_pack, W1.astype(f32), b1r)

    z2 = _sc_edge_pass(y1, src_pack, dst_pack, zerosD)

    b2r = b2.astype(f32).reshape(1, D)
    fcb1r = fcb1.astype(f32).reshape(1, 16)
    fcw2p = jnp.zeros((16, 16), f32).at[:, :C].set(fcW2.astype(f32))
    fcb2p = jnp.zeros((1, 16), f32).at[0, :C].set(fcb2.astype(f32))
    dw1 = dW[:D].astype(f32)
    dw2 = dW[D:2 * D].astype(f32)
    dwbp = jnp.zeros((16, 2), f32).at[:C].set(dW[2 * D:].astype(f32))
    dbr = db.astype(f32).reshape(1, 2)
    lab = labels_s.astype(jnp.int32).reshape(G, 1)

    _, loss, closs, dloss = _tc_final(
        z2, y1, deg16, batch_pack, W2.astype(f32), b2r, ps1, cnt,
        fcW1.astype(f32), fcb1r, fcw2p, fcb2p, dw1, dw2, dwbp, dbr, lab)

    return (loss.reshape(()), closs.reshape(()), dloss.reshape(()))


# speedup vs baseline: 19.6052x; 1.0232x over previous
import functools

import jax
import jax.numpy as jnp
from jax import lax
from jax.experimental import pallas as pl
from jax.experimental.pallas import tpu as pltpu
from jax.experimental.pallas import tpu_sc as plsc

N = 10000
D = 128
G = 256
C = 10
E = 320000

NPAD = 10112
RPT = NPAD // 16
CH = 128
NCH = 160
ET = NCH * CH
EPAD = 16 * ET
IB = 32
NIB = NCH // IB
RB = 2 * NPAD // 8
GSEG = 2 * G


@functools.cache
def _sc_kernels():
    mesh = plsc.VectorSubcoreMesh(core_axis_name="c", subcore_axis_name="s")

    @functools.partial(
        pl.kernel,
        mesh=mesh,
        out_type=jax.ShapeDtypeStruct((2 * NPAD, D), jnp.float32),
        scratch_types=[
            pltpu.VMEM((IB, CH), jnp.int32),
            pltpu.VMEM((CH, D), jnp.float32),
            pltpu.VMEM_SHARED((NPAD, D), jnp.float32),
            pltpu.SemaphoreType.DMA,
        ],
    )
    def sc_degree(dst_hbm, ones_hbm, zeros_hbm, out_hbm, dst_v, ones_v, acc,
                  sem):
        c = lax.axis_index("c")
        s = lax.axis_index("s")
        pltpu.sync_copy(zeros_hbm, acc.at[pl.ds(s * RPT, RPT)])
        pltpu.sync_copy(ones_hbm, ones_v)
        plsc.subcore_barrier()

        def block(bi, carry):
            base = (c * 16 + s) * NCH + bi * IB
            pltpu.sync_copy(dst_hbm.at[pl.ds(base, IB)], dst_v)

            def fire(j, carry2):
                pltpu.async_copy(ones_v, acc.at[dst_v.at[j]], sem, add=True)
                return carry2

            lax.fori_loop(0, IB, fire, 0)

            def drain(j, carry2):
                pltpu.make_async_copy(ones_v, acc.at[dst_v.at[j]], sem).wait()
                return carry2

            lax.fori_loop(0, IB, drain, 0)
            return carry

        lax.fori_loop(0, NIB, block, 0)
        plsc.subcore_barrier()
        pltpu.sync_copy(acc.at[pl.ds(s * RPT, RPT)],
                        out_hbm.at[pl.ds(c * NPAD + s * RPT, RPT)])

    @functools.partial(
        pl.kernel,
        mesh=mesh,
        out_type=jax.ShapeDtypeStruct((2 * NPAD, D), jnp.float32),
        scratch_types=[
            pltpu.VMEM((IB, CH), jnp.int32),
            pltpu.VMEM((IB, CH), jnp.int32),
            pltpu.VMEM((CH, D), jnp.float32),
            pltpu.VMEM((CH, D), jnp.float32),
            pltpu.VMEM_SHARED((NPAD, D), jnp.float32),
            pltpu.SemaphoreType.DMA,
            pltpu.SemaphoreType.DMA,
            pltpu.SemaphoreType.DMA,
            pltpu.SemaphoreType.DMA,
        ],
    )
    def sc_edge_pass(y_hbm, src_hbm, dst_hbm, zeros_hbm, out_hbm,
                     src_v, dst_v, rows0, rows1, acc, gs0, gs1, ss0, ss1):
        rows = [rows0, rows1]
        gsems = [gs0, gs1]
        ssems = [ss0, ss1]
        c = lax.axis_index("c")
        s = lax.axis_index("s")
        pltpu.sync_copy(zeros_hbm, acc.at[pl.ds(s * RPT, RPT)])
        plsc.subcore_barrier()

        def block(bi, carry):
            base = (c * 16 + s) * NCH + bi * IB
            pltpu.sync_copy(src_hbm.at[pl.ds(base, IB)], src_v)
            pltpu.sync_copy(dst_hbm.at[pl.ds(base, IB)], dst_v)

            def pair(i, carry2):
                for b in range(2):
                    j = 2 * i + b

                    @pl.when(i > 0)
                    def _():
                        pltpu.make_async_copy(
                            rows[b], acc.at[dst_v.at[j - 2]], ssems[b]).wait()

                    pltpu.async_copy(y_hbm.at[src_v.at[j]], rows[b], gsems[b])
                for b in range(2):
                    j = 2 * i + b
                    pltpu.make_async_copy(y_hbm.at[src_v.at[j]], rows[b],
                                          gsems[b]).wait()
                    pltpu.async_copy(rows[b], acc.at[dst_v.at[j]], ssems[b],
                                     add=True)
                return carry2

            lax.fori_loop(0, IB // 2, pair, 0)
            for b in range(2):
                pltpu.make_async_copy(rows[b], acc.at[dst_v.at[IB - 2 + b]],
                                      ssems[b]).wait()
            return carry

        lax.fori_loop(0, NIB, block, 0)
        plsc.subcore_barrier()
        pltpu.sync_copy(acc.at[pl.ds(s * RPT, RPT)],
                        out_hbm.at[pl.ds(c * NPAD + s * RPT, RPT)])

    return sc_degree, sc_edge_pass


def _sc_degree(dst_pack, onesD, zerosD):
    return _sc_kernels()[0](dst_pack, onesD, zerosD)


def _sc_edge_pass(y, src_pack, dst_pack, zerosD):
    return _sc_kernels()[1](y, src_pack, dst_pack, zerosD)


def _scale_body(x_ref, deg_ref, y_ref):
    dinv = lax.rsqrt(deg_ref[:, 0:1] + 1.0)
    y_ref[...] = x_ref[...] * dinv


def _mid_body(z_ref, y0_ref, deg_ref, b_ref, w_ref, bias_ref,
              y1_ref, ps_ref, cnt_ref):
    i = pl.program_id(0)
    dinv = lax.rsqrt(deg_ref[:, 0:1] + 1.0)
    a1 = dinv * (z_ref[...] + y0_ref[...])
    f1 = jnp.dot(a1, w_ref[...], preferred_element_type=jnp.float32) + bias_ref[...]
    y1_ref[...] = dinv * f1
    onehot = (b_ref[...] == lax.broadcasted_iota(jnp.int32, (RB, GSEG), 1)
              ).astype(jnp.float32)
    dn = (((0,), (0,)), ((), ()))
    blk_ps = lax.dot_general(onehot, f1, dn, preferred_element_type=jnp.float32)
    blk_cnt = lax.dot_general(onehot, jnp.ones_like(f1), dn,
                              preferred_element_type=jnp.float32)

    @pl.when(i == 0)
    def _():
        ps_ref[...] = blk_ps
        cnt_ref[...] = blk_cnt

    @pl.when(i > 0)
    def _():
        ps_ref[...] += blk_ps
        cnt_ref[...] += blk_cnt


def _final_body(z_ref, y1_ref, deg_ref, b_ref, w_ref, bias_ref,
                ps1_ref, cnt_ref, fcw1_ref, fcb1_ref, fcw2_ref, fcb2_ref,
                dw1_ref, dw2_ref, dwb_ref, db_ref, lab_ref,
                ps2_ref, loss_ref, closs_ref, dloss_ref):
    i = pl.program_id(0)
    dinv = lax.rsqrt(deg_ref[:, 0:1] + 1.0)
    a2 = dinv * (z_ref[...] + y1_ref[...])
    f2 = jnp.dot(a2, w_ref[...], preferred_element_type=jnp.float32) + bias_ref[...]
    onehot = (b_ref[...] == lax.broadcasted_iota(jnp.int32, (RB, GSEG), 1)
              ).astype(jnp.float32)
    dn = (((0,), (0,)), ((), ()))
    blk_ps = lax.dot_general(onehot, f2, dn, preferred_element_type=jnp.float32)

    @pl.when(i == 0)
    def _():
        ps2_ref[...] = blk_ps

    @pl.when(i > 0)
    def _():
        ps2_ref[...] += blk_ps

    @pl.when(i == 7)
    def _():
        cnt = jnp.maximum(cnt_ref[...], 1.0)
        p1 = ps1_ref[...] / cnt
        p2 = ps2_ref[...] / cnt
        p1s, p1t = p1[:G], p1[G:]
        p2s, p2t = p2[:G], p2[G:]

        def head(p):
            h = jnp.maximum(
                jnp.dot(p, fcw1_ref[...], preferred_element_type=jnp.float32)
                + fcb1_ref[...], 0.0)
            return (jnp.dot(h, fcw2_ref[...], preferred_element_type=jnp.float32)
                    + fcb2_ref[...])

        fsb = head(p2s)
        ftb = head(p2t)
        y16 = (lab_ref[...] == lax.broadcasted_iota(jnp.int32, (G, 16), 1)
               ).astype(jnp.float32)
        z10 = fsb[:, :C]
        y10 = y16[:, :C]
        closs = jnp.mean(jnp.maximum(z10, 0.0) - z10 * y10
                         + jnp.log1p(jnp.exp(-jnp.abs(z10))))

        def dom_logits(pa, pb, fb):
            return (jnp.dot(pa, dw1_ref[...], preferred_element_type=jnp.float32)
                    + jnp.dot(pb, dw2_ref[...], preferred_element_type=jnp.float32)
                    + jnp.dot(fb, dwb_ref[...], preferred_element_type=jnp.float32)
                    + db_ref[...])

        dp_s = dom_logits(p1s, p2s, fsb)
        dp_t = dom_logits(p1t, p2t, ftb)

        def lse2(dp):
            m = jnp.maximum(dp[:, 0:1], dp[:, 1:2])
            return m + jnp.log(jnp.exp(dp[:, 0:1] - m) + jnp.exp(dp[:, 1:2] - m))

        dloss = -(jnp.sum(dp_s[:, 0:1] - lse2(dp_s))
                  + jnp.sum(dp_t[:, 1:2] - lse2(dp_t))) / (2.0 * G)
        loss = closs + 0.01 * dloss
        loss_ref[...] = loss.reshape(1, 1)
        closs_ref[...] = closs.reshape(1, 1)
        dloss_ref[...] = dloss.reshape(1, 1)


def _row_spec(w):
    return pl.BlockSpec((RB, w), lambda i: (i, 0))


def _const_spec(shape):
    return pl.BlockSpec(shape, lambda i: tuple(0 for _ in shape))


def _tc_scale(x_pack, deg16):
    return pl.pallas_call(
        _scale_body,
        grid=(8,),
        in_specs=[_row_spec(D), _row_spec(D)],
        out_specs=_row_spec(D),
        out_shape=jax.ShapeDtypeStruct((2 * NPAD, D), jnp.float32),
    )(x_pack, deg16)


def _tc_mid(z1, y0, deg16, batch_pack, w1, b1r):
    return pl.pallas_call(
        _mid_body,
        grid=(8,),
        in_specs=[_row_spec(D), _row_spec(D), _row_spec(D), _row_spec(1),
                  _const_spec((D, D)), _const_spec((1, D))],
        out_specs=[_row_spec(D), _const_spec((GSEG, D)), _const_spec((GSEG, D))],
        out_shape=[jax.ShapeDtypeStruct((2 * NPAD, D), jnp.float32),
                   jax.ShapeDtypeStruct((GSEG, D), jnp.float32),
                   jax.ShapeDtypeStruct((GSEG, D), jnp.float32)],
    )(z1, y0, deg16, batch_pack, w1, b1r)


def _tc_final(z2, y1, deg16, batch_pack, w2, b2r, ps1, cnt,
              fcw1, fcb1r, fcw2p, fcb2p, dw1, dw2, dwbp, dbr, lab):
    return pl.pallas_call(
        _final_body,
        grid=(8,),
        in_specs=[_row_spec(D), _row_spec(D), _row_spec(D), _row_spec(1),
                  _const_spec((D, D)), _const_spec((1, D)),
                  _const_spec((GSEG, D)), _const_spec((GSEG, D)),
                  _const_spec((D, 16)), _const_spec((1, 16)),
                  _const_spec((16, 16)), _const_spec((1, 16)),
                  _const_spec((D, 2)), _const_spec((D, 2)),
                  _const_spec((16, 2)), _const_spec((1, 2)),
                  _const_spec((G, 1))],
        out_specs=[_const_spec((GSEG, D)), _const_spec((1, 1)),
                   _const_spec((1, 1)), _const_spec((1, 1))],
        out_shape=[jax.ShapeDtypeStruct((GSEG, D), jnp.float32),
                   jax.ShapeDtypeStruct((1, 1), jnp.float32),
                   jax.ShapeDtypeStruct((1, 1), jnp.float32),
                   jax.ShapeDtypeStruct((1, 1), jnp.float32)],
    )(z2, y1, deg16, batch_pack, w2, b2r, ps1, cnt,
      fcw1, fcb1r, fcw2p, fcb2p, dw1, dw2, dwbp, dbr, lab)


def _pad_edges(idx, row_offset):
    pad = N + (jnp.arange(EPAD - E, dtype=jnp.int32) % 16)
    full = jnp.concatenate([idx.astype(jnp.int32), pad]) + row_offset
    return full.reshape(16 * NCH, CH)


def kernel(x_s, x_t, W1, b1, W2, b2, fcW1, fcb1, fcW2, fcb2, dW, db,
           edge_index_s, edge_index_t, batch_s, batch_t, labels_s):
    f32 = jnp.float32
    zpad = jnp.zeros((NPAD - N, D), f32)
    x_pack = jnp.concatenate([x_s.astype(f32), zpad, x_t.astype(f32), zpad])

    src_pack = jnp.concatenate([_pad_edges(edge_index_s[0], 0),
                                _pad_edges(edge_index_t[0], NPAD)])
    dst_pack = jnp.concatenate([_pad_edges(edge_index_s[1], 0),
                                _pad_edges(edge_index_t[1], 0)])

    bpad = jnp.full((NPAD - N,), GSEG, jnp.int32)
    batch_pack = jnp.concatenate([
        batch_s.astype(jnp.int32), bpad,
        batch_t.astype(jnp.int32) + G, bpad]).reshape(2 * NPAD, 1)

    onesD = jnp.ones((CH, D), f32)
    zerosD = jnp.zeros((RPT, D), f32)

    deg16 = _sc_degree(dst_pack, onesD, zerosD)
    y0 = _tc_scale(x_pack, deg16)
    z1 = _sc_edge_pass(y0, src_pack, dst_pack, zerosD)

    b1r = b1.astype(f32).reshape(1, D)
    y1, ps1, cnt = _tc_mid(z1, y0, deg16, batch_pack, W1.astype(f32), b1r)

    z2 = _sc_edge_pass(y1, src_pack, dst_pack, zerosD)

    b2r = b2.astype(f32).reshape(1, D)
    fcb1r = fcb1.astype(f32).reshape(1, 16)
    fcw2p = jnp.zeros((16, 16), f32).at[:, :C].set(fcW2.astype(f32))
    fcb2p = jnp.zeros((1, 16), f32).at[0, :C].set(fcb2.astype(f32))
    dw1 = dW[:D].astype(f32)
    dw2 = dW[D:2 * D].astype(f32)
    dwbp = jnp.zeros((16, 2), f32).at[:C].set(dW[2 * D:].astype(f32))
    dbr = db.astype(f32).reshape(1, 2)
    lab = labels_s.astype(jnp.int32).reshape(G, 1)

    _, loss, closs, dloss = _tc_final(
        z2, y1, deg16, batch_pack, W2.astype(f32), b2r, ps1, cnt,
        fcW1.astype(f32), fcb1r, fcw2p, fcb2p, dw1, dw2, dwbp, dbr, lab)

    return (loss.reshape(()), closs.reshape(()), dloss.reshape(()))
